# baseline (speedup 1.0000x reference)
import functools

import jax
import jax.numpy as jnp
from jax import lax
from jax.experimental import pallas as pl
from jax.experimental.pallas import tpu as pltpu
from jax.experimental.pallas import tpu_sc as plsc

N_RANK = 50
N_PAIR = 25
PACK_COLS = 128
IDS_PER_ROW = 4
BATCH = 16384
N_USERS_P = 1000000
NUM_CORES = 2
NUM_SUBCORES = 16
NW = NUM_CORES * NUM_SUBCORES
B_PER_W = BATCH // NW
LANES = 16
CHUNK = 128
N_CHUNKS = B_PER_W // CHUNK
GROUPS = CHUNK // LANES


@functools.partial(
    pl.kernel,
    out_type=jax.ShapeDtypeStruct((BATCH,), jnp.float32),
    mesh=plsc.VectorSubcoreMesh(core_axis_name="c", subcore_axis_name="s"),
    compiler_params=pltpu.CompilerParams(needs_layout_passes=False),
    scratch_types=[
        pltpu.VMEM((B_PER_W,), jnp.int32),
        pltpu.VMEM((B_PER_W,), jnp.int32),
        pltpu.VMEM((CHUNK,), jnp.int32),
        pltpu.VMEM((CHUNK,), jnp.int32),
        pltpu.VMEM((CHUNK, PACK_COLS), jnp.int32),
        pltpu.VMEM((CHUNK, PACK_COLS), jnp.int32),
        pltpu.VMEM((B_PER_W,), jnp.float32),
        pltpu.VMEM((B_PER_W,), jnp.float32),
        pltpu.VMEM((LANES,), jnp.float32),
        pltpu.VMEM((B_PER_W,), jnp.float32),
        pltpu.SemaphoreType.DMA,
        pltpu.SemaphoreType.DMA,
        pltpu.SemaphoreType.DMA,
        pltpu.SemaphoreType.DMA,
    ],
)
def _lfm_sc(uid_hbm, iid_hbm, p_hbm, q_hbm, mu_hbm, ub_hbm, ib_hbm,
            out_hbm, uidx_v, iidx_v, urow_v, irow_v, prows_v, qrows_v,
            ubias_v, ibias_v, mu_v, out_v, sem_p, sem_q, sem_ub, sem_ib):
    wid = lax.axis_index("s") * NUM_CORES + lax.axis_index("c")
    base = wid * B_PER_W

    pltpu.sync_copy(uid_hbm.at[pl.ds(base, B_PER_W)], uidx_v)
    pltpu.sync_copy(iid_hbm.at[pl.ds(base, B_PER_W)], iidx_v)
    pltpu.sync_copy(mu_hbm, mu_v.at[pl.ds(0, 1)])

    cp_ub = pltpu.async_copy(ub_hbm.at[uidx_v], ubias_v, sem_ub)
    cp_ib = pltpu.async_copy(ib_hbm.at[iidx_v], ibias_v, sem_ib)

    mu_vec = jnp.broadcast_to(mu_v[...][0], (LANES,))
    lane_iota = lax.iota(jnp.int32, LANES)
    himask = jnp.full((LANES,), -65536, jnp.int32)

    def chunk_body(c, carry):
        cbase = c * CHUNK

        def rowidx(g, carry2):
            sl_src = pl.ds(cbase + g * LANES, LANES)
            sl_dst = pl.ds(g * LANES, LANES)
            urow_v[sl_dst] = lax.shift_right_logical(uidx_v[sl_src], 2)
            irow_v[sl_dst] = lax.shift_right_logical(iidx_v[sl_src], 2)
            return carry2

        lax.fori_loop(0, GROUPS, rowidx, 0)

        cp_p = pltpu.async_copy(p_hbm.at[urow_v], prows_v, sem_p)
        cp_q = pltpu.async_copy(q_hbm.at[irow_v], qrows_v, sem_q)
        cp_p.wait()
        cp_q.wait()

        def group_body(g, carry2):
            row0 = g * LANES
            rid = row0 + lane_iota
            uvec = uidx_v[pl.ds(cbase + row0, LANES)]
            ivec = iidx_v[pl.ds(cbase + row0, LANES)]
            ucol = lax.shift_left(jnp.bitwise_and(uvec, 3), 5)
            icol = lax.shift_left(jnp.bitwise_and(ivec, 3), 5)
            acc = jnp.zeros((LANES,), jnp.float32)
            for k in range(N_PAIR):
                pk = plsc.load_gather(prows_v, [rid, ucol + k])
                qk = plsc.load_gather(qrows_v, [rid, icol + k])
                plo = plsc.bitcast(lax.shift_left(pk, 16), jnp.float32)
                qlo = plsc.bitcast(lax.shift_left(qk, 16), jnp.float32)
                phi = plsc.bitcast(jnp.bitwise_and(pk, himask), jnp.float32)
                qhi = plsc.bitcast(jnp.bitwise_and(qk, himask), jnp.float32)
                acc = acc + plo * qlo + phi * qhi
            out_v[pl.ds(cbase + row0, LANES)] = acc
            return carry2

        lax.fori_loop(0, GROUPS, group_body, 0)
        return carry

    lax.fori_loop(0, N_CHUNKS, chunk_body, 0)

    cp_ub.wait()
    cp_ib.wait()

    def bias_body(g, carry):
        row0 = g * LANES
        sl = pl.ds(row0, LANES)
        out_v[sl] = out_v[sl] + ubias_v[sl] + ibias_v[sl] + mu_vec
        return carry

    lax.fori_loop(0, B_PER_W // LANES, bias_body, 0)
    pltpu.sync_copy(out_v, out_hbm.at[pl.ds(base, B_PER_W)])


def _pack(table):
    n = table.shape[0]
    padded = jnp.pad(table, ((0, 0), (0, 64 - N_RANK)))
    packed = jax.lax.bitcast_convert_type(
        padded.astype(jnp.bfloat16).reshape(n, 32, 2), jnp.int32)
    return packed.reshape(n // IDS_PER_ROW, PACK_COLS)


def kernel(user_ids, item_ids, P, Q, mu, user_bias, item_bias):
    return _lfm_sc(user_ids.astype(jnp.int32), item_ids.astype(jnp.int32),
                   _pack(P), _pack(Q), mu, user_bias, item_bias)


# speedup vs baseline: 1.4232x; 1.4232x over previous
import functools

import jax
import jax.numpy as jnp
from jax import lax
from jax.experimental import pallas as pl
from jax.experimental.pallas import tpu as pltpu
from jax.experimental.pallas import tpu_sc as plsc

N_RANK = 50
PACK_COLS = 128
IDS_PER_ROW = 2
BATCH = 16384
N_USERS_P = 1000000
NUM_CORES = 2
NUM_SUBCORES = 16
NW = NUM_CORES * NUM_SUBCORES
B_PER_W = BATCH // NW
LANES = 16
CHUNK = 128
N_CHUNKS = B_PER_W // CHUNK
GROUPS = CHUNK // LANES


@functools.partial(
    pl.kernel,
    out_type=jax.ShapeDtypeStruct((BATCH,), jnp.float32),
    mesh=plsc.VectorSubcoreMesh(core_axis_name="c", subcore_axis_name="s"),
    compiler_params=pltpu.CompilerParams(needs_layout_passes=False),
    scratch_types=[
        pltpu.VMEM((B_PER_W,), jnp.int32),
        pltpu.VMEM((B_PER_W,), jnp.int32),
        pltpu.VMEM((CHUNK,), jnp.int32),
        pltpu.VMEM((CHUNK,), jnp.int32),
        pltpu.VMEM((CHUNK, PACK_COLS), jnp.float32),
        pltpu.VMEM((CHUNK, PACK_COLS), jnp.float32),
        pltpu.VMEM((B_PER_W,), jnp.float32),
        pltpu.VMEM((B_PER_W,), jnp.float32),
        pltpu.VMEM((LANES,), jnp.float32),
        pltpu.VMEM((B_PER_W,), jnp.float32),
        pltpu.SemaphoreType.DMA,
        pltpu.SemaphoreType.DMA,
        pltpu.SemaphoreType.DMA,
        pltpu.SemaphoreType.DMA,
    ],
)
def _lfm_sc(uid_hbm, iid_hbm, p_hbm, q_hbm, mu_hbm, ub_hbm, ib_hbm,
            out_hbm, uidx_v, iidx_v, urow_v, irow_v, prows_v, qrows_v,
            ubias_v, ibias_v, mu_v, out_v, sem_p, sem_q, sem_ub, sem_ib):
    wid = lax.axis_index("s") * NUM_CORES + lax.axis_index("c")
    base = wid * B_PER_W

    pltpu.sync_copy(uid_hbm.at[pl.ds(base, B_PER_W)], uidx_v)
    pltpu.sync_copy(iid_hbm.at[pl.ds(base, B_PER_W)], iidx_v)
    pltpu.sync_copy(mu_hbm, mu_v.at[pl.ds(0, 1)])

    cp_ub = pltpu.async_copy(ub_hbm.at[uidx_v], ubias_v, sem_ub)
    cp_ib = pltpu.async_copy(ib_hbm.at[iidx_v], ibias_v, sem_ib)

    mu_vec = jnp.broadcast_to(mu_v[...][0], (LANES,))
    lane_iota = lax.iota(jnp.int32, LANES)

    def chunk_body(c, carry):
        cbase = c * CHUNK

        def rowidx(g, carry2):
            sl_src = pl.ds(cbase + g * LANES, LANES)
            sl_dst = pl.ds(g * LANES, LANES)
            urow_v[sl_dst] = lax.shift_right_logical(uidx_v[sl_src], 1)
            irow_v[sl_dst] = lax.shift_right_logical(iidx_v[sl_src], 1)
            return carry2

        lax.fori_loop(0, GROUPS, rowidx, 0)

        cp_p = pltpu.async_copy(p_hbm.at[urow_v], prows_v, sem_p)
        cp_q = pltpu.async_copy(q_hbm.at[irow_v], qrows_v, sem_q)
        cp_p.wait()
        cp_q.wait()

        def group_body(g, carry2):
            row0 = g * LANES
            rid = row0 + lane_iota
            uvec = uidx_v[pl.ds(cbase + row0, LANES)]
            ivec = iidx_v[pl.ds(cbase + row0, LANES)]
            ucol = lax.shift_left(jnp.bitwise_and(uvec, 1), 6)
            icol = lax.shift_left(jnp.bitwise_and(ivec, 1), 6)
            acc = jnp.zeros((LANES,), jnp.float32)
            for r in range(N_RANK):
                pv = plsc.load_gather(prows_v, [rid, ucol + r])
                qv = plsc.load_gather(qrows_v, [rid, icol + r])
                acc = acc + pv * qv
            out_v[pl.ds(cbase + row0, LANES)] = acc
            return carry2

        lax.fori_loop(0, GROUPS, group_body, 0)
        return carry

    lax.fori_loop(0, N_CHUNKS, chunk_body, 0)

    cp_ub.wait()
    cp_ib.wait()

    def bias_body(g, carry):
        row0 = g * LANES
        sl = pl.ds(row0, LANES)
        out_v[sl] = out_v[sl] + ubias_v[sl] + ibias_v[sl] + mu_vec
        return carry

    lax.fori_loop(0, B_PER_W // LANES, bias_body, 0)
    pltpu.sync_copy(out_v, out_hbm.at[pl.ds(base, B_PER_W)])


def _pack(table):
    n = table.shape[0]
    padded = jnp.pad(table, ((0, 0), (0, 64 - N_RANK)))
    return padded.reshape(n // IDS_PER_ROW, PACK_COLS)


def kernel(user_ids, item_ids, P, Q, mu, user_bias, item_bias):
    return _lfm_sc(user_ids.astype(jnp.int32), item_ids.astype(jnp.int32),
                   _pack(P), _pack(Q), mu, user_bias, item_bias)


# speedup vs baseline: 4.8058x; 3.3766x over previous
import functools

import jax
import jax.numpy as jnp
from jax import lax
from jax.experimental import pallas as pl
from jax.experimental.pallas import tpu as pltpu
from jax.experimental.pallas import tpu_sc as plsc

N_RANK = 50
BATCH = 16384
NUM_CORES = 2
NUM_SUBCORES = 16
NW = NUM_CORES * NUM_SUBCORES
B_PER_W = BATCH // NW
LANES = 16
ROW_PAD = 128
CHUNK = 128
N_CHUNKS = B_PER_W // CHUNK
GROUPS = CHUNK // LANES


@functools.partial(
    pl.kernel,
    out_type=jax.ShapeDtypeStruct((BATCH,), jnp.float32),
    mesh=plsc.VectorSubcoreMesh(core_axis_name="c", subcore_axis_name="s"),
    compiler_params=pltpu.CompilerParams(needs_layout_passes=False),
    scratch_types=[
        pltpu.VMEM((B_PER_W,), jnp.int32),
        pltpu.VMEM((B_PER_W,), jnp.int32),
        pltpu.VMEM((CHUNK, ROW_PAD), jnp.float32),
        pltpu.VMEM((CHUNK, ROW_PAD), jnp.float32),
        pltpu.VMEM((B_PER_W,), jnp.float32),
        pltpu.VMEM((B_PER_W,), jnp.float32),
        pltpu.VMEM((LANES,), jnp.float32),
        pltpu.VMEM((B_PER_W,), jnp.float32),
        pltpu.SemaphoreType.DMA,
        pltpu.SemaphoreType.DMA,
        pltpu.SemaphoreType.DMA,
        pltpu.SemaphoreType.DMA,
    ],
)
def _lfm_sc(uid_hbm, iid_hbm, p_hbm, q_hbm, mu_hbm, ub_hbm, ib_hbm,
            out_hbm, uidx_v, iidx_v, prows_v, qrows_v, ubias_v, ibias_v,
            mu_v, out_v, sem_p, sem_q, sem_ub, sem_ib):
    wid = lax.axis_index("s") * NUM_CORES + lax.axis_index("c")
    base = wid * B_PER_W

    pltpu.sync_copy(uid_hbm.at[pl.ds(base, B_PER_W)], uidx_v)
    pltpu.sync_copy(iid_hbm.at[pl.ds(base, B_PER_W)], iidx_v)
    pltpu.sync_copy(mu_hbm, mu_v.at[pl.ds(0, 1)])

    cp_ub = pltpu.async_copy(ub_hbm.at[uidx_v], ubias_v, sem_ub)
    cp_ib = pltpu.async_copy(ib_hbm.at[iidx_v], ibias_v, sem_ib)

    mu_vec = jnp.broadcast_to(mu_v[...][0], (LANES,))
    lane_iota = lax.iota(jnp.int32, LANES)

    def chunk_body(c, carry):
        cbase = c * CHUNK

        def fire(g, carry2):
            jbase = g * LANES
            uvec = uidx_v[pl.ds(cbase + jbase, LANES)]
            ivec = iidx_v[pl.ds(cbase + jbase, LANES)]
            for j in range(LANES):
                pltpu.async_copy(p_hbm.at[uvec[j]],
                                 prows_v.at[jbase + j, pl.ds(0, N_RANK)],
                                 sem_p)
                pltpu.async_copy(q_hbm.at[ivec[j]],
                                 qrows_v.at[jbase + j, pl.ds(0, N_RANK)],
                                 sem_q)
            return carry2

        lax.fori_loop(0, GROUPS, fire, 0)

        def drain(j, carry2):
            pltpu.make_async_copy(
                p_hbm.at[0], prows_v.at[j, pl.ds(0, N_RANK)], sem_p).wait()
            pltpu.make_async_copy(
                q_hbm.at[0], qrows_v.at[j, pl.ds(0, N_RANK)], sem_q).wait()
            return carry2

        lax.fori_loop(0, CHUNK, drain, 0)

        def group_body(g, carry2):
            row0 = g * LANES
            rid = row0 + lane_iota
            acc = jnp.zeros((LANES,), jnp.float32)
            for r in range(N_RANK):
                cid = jnp.full((LANES,), r, jnp.int32)
                pv = plsc.load_gather(prows_v, [rid, cid])
                qv = plsc.load_gather(qrows_v, [rid, cid])
                acc = acc + pv * qv
            out_v[pl.ds(cbase + row0, LANES)] = acc
            return carry2

        lax.fori_loop(0, GROUPS, group_body, 0)
        return carry

    lax.fori_loop(0, N_CHUNKS, chunk_body, 0)

    cp_ub.wait()
    cp_ib.wait()

    def bias_body(g, carry):
        row0 = g * LANES
        sl = pl.ds(row0, LANES)
        out_v[sl] = out_v[sl] + ubias_v[sl] + ibias_v[sl] + mu_vec
        return carry

    lax.fori_loop(0, B_PER_W // LANES, bias_body, 0)
    pltpu.sync_copy(out_v, out_hbm.at[pl.ds(base, B_PER_W)])


def kernel(user_ids, item_ids, P, Q, mu, user_bias, item_bias):
    return _lfm_sc(user_ids.astype(jnp.int32), item_ids.astype(jnp.int32),
                   P, Q, mu, user_bias, item_bias)
